# baseline (speedup 1.0000x reference)
import functools

import jax
import jax.numpy as jnp
from jax import lax
from jax.experimental import pallas as pl
from jax.experimental.pallas import tpu as pltpu
from jax.experimental.pallas import tpu_sc as plsc

B, Q, T, C, K = 128, 900, 256, 91, 300
CPAD = 128
NB = Q * CPAD
QPAD = 1024
CAP = 512
CCAP = 4096
NEG1_BITS = -1082130432


def _prob_body(logits_ref, pm_ref, prob_ref, qmax_ref):
    x = logits_ref[0]
    sig = 1.0 / (1.0 + jnp.exp(-x))
    pm = pm_ref[...]
    prob = jax.lax.dot_general(
        sig, pm, (((1,), (1,)), ((), ())),
        preferred_element_type=jnp.float32)
    col = lax.broadcasted_iota(jnp.int32, (Q, CPAD), 1)
    bits = lax.bitcast_convert_type(prob, jnp.int32)
    bits = jnp.where(col < C, bits, NEG1_BITS)
    prob_ref[0] = bits
    qmax = jnp.max(bits, axis=1)
    qmax_ref[0, 0] = jnp.concatenate(
        [qmax, jnp.full((QPAD - Q,), NEG1_BITS, jnp.int32)])


def _compute_prob(pred_logits, positive_map):
    pm96 = jnp.concatenate(
        [positive_map, jnp.zeros((CPAD - C, T), jnp.float32)], axis=0)
    return pl.pallas_call(
        _prob_body,
        grid=(B,),
        in_specs=[
            pl.BlockSpec((1, Q, T), lambda b: (b, 0, 0)),
            pl.BlockSpec((CPAD, T), lambda b: (0, 0)),
        ],
        out_specs=[pl.BlockSpec((1, Q, CPAD), lambda b: (b, 0, 0)),
                   pl.BlockSpec((1, 1, QPAD), lambda b: (b, 0, 0))],
        out_shape=[jax.ShapeDtypeStruct((B, Q, CPAD), jnp.int32),
                   jax.ShapeDtypeStruct((B, 1, QPAD), jnp.int32)],
    )(pred_logits, pm96)


def _selector_body(nb, k, cap, ccap, qpad, rows_per_w, nc, prob_hbm,
                   qmax_hbm, vals_hbm, idx_hbm, row_v, qm_v, cand_k, cand_i,
                   ov_v, oi_v):
    nv = nb // 16
    UN = 8
    lane = lax.broadcasted_iota(jnp.int32, (16,), 0)
    zeros = jnp.zeros((16,), jnp.int32)
    neg1 = jnp.full((16,), NEG1_BITS, jnp.int32)
    true16 = jnp.ones((16,), jnp.bool_)
    wid = lax.axis_index("s") * nc + lax.axis_index("c")

    def select_row(row, _):
        pltpu.sync_copy(prob_hbm.at[row], row_v)
        pltpu.sync_copy(qmax_hbm.at[row], qm_v)

        def bs_m(j, cur):
            t = cur | lax.shift_left(jnp.int32(1), 30 - j)

            def sweep(i, acc):
                return acc + (qm_v[pl.ds(i * 16, 16)] >= t).astype(jnp.int32)
            cnt = jnp.sum(lax.fori_loop(0, qpad // 16, sweep, zeros))
            return jnp.where(cnt >= k, t, cur)
        m = lax.fori_loop(0, 31, bs_m, jnp.int32(0))

        def g(t, ptrv):
            kks = []
            ms = []
            for u in range(UN):
                i = t * UN + u
                kk = row_v[pl.ds(i * 16, 16)]
                kks.append(kk)
                ms.append(kk >= m)
            anym = ms[0]
            for u in range(1, UN):
                anym = anym | ms[u]

            def do_store():
                pv = ptrv
                for u in range(UN):
                    pos = pv + plsc.cumsum(ms[u].astype(jnp.int32)) - 1
                    pos = jnp.minimum(pos, ccap + 16)
                    plsc.store_scatter(cand_k, [pos], kks[u], mask=ms[u])
                    plsc.store_scatter(cand_i, [pos], (t * UN + u) * 16 + lane,
                                       mask=ms[u])
                    pv = pv + plsc.all_reduce_population_count(ms[u])
                return pv
            return lax.cond(jnp.any(anym), do_store, lambda: ptrv)
        ptrv = lax.fori_loop(0, nv // UN, g, zeros)
        n_cand = jnp.max(ptrv)

        def initf(j, _):
            ov_v[pl.ds(j * 16, 16)] = neg1
            oi_v[pl.ds(j * 16, 16)] = zeros
            return _
        lax.fori_loop(0, (cap + 16) // 16, initf, 0)

        def finish(src_k, load_idx, ntrips):
            def bs(j, cur):
                t = cur | lax.shift_left(jnp.int32(1), 30 - j)

                def sweep(i, acc):
                    kk = src_k[pl.ds(i * 16, 16)]
                    return acc + (kk >= t).astype(jnp.int32)
                cnt = jnp.sum(lax.fori_loop(0, ntrips, sweep, zeros))
                return jnp.where(cnt >= k, t, cur)
            vk = lax.fori_loop(0, 31, bs, jnp.int32(0))

            def sweep2(i, acc):
                kk = src_k[pl.ds(i * 16, 16)]
                return acc + (kk > vk).astype(jnp.int32)
            n_gt = jnp.sum(lax.fori_loop(0, ntrips, sweep2, zeros))
            eqb = k - n_gt

            def ext(i, carry):
                ptrv, eqt = carry
                kk = src_k[pl.ds(i * 16, 16)]
                iv = load_idx(i)
                gt = kk > vk
                eq = kk == vk
                eqc = plsc.cumsum(eq.astype(jnp.int32))
                acc_eq = eq & ((eqc + eqt) <= eqb)
                accept = gt | acc_eq
                pos = ptrv + plsc.cumsum(accept.astype(jnp.int32)) - 1
                plsc.store_scatter(ov_v, [pos], kk, mask=accept)
                plsc.store_scatter(oi_v, [pos], iv, mask=accept)
                ptrv = ptrv + plsc.all_reduce_population_count(accept)
                eqt = eqt + plsc.all_reduce_population_count(acc_eq)
                return ptrv, eqt
            lax.fori_loop(0, ntrips, ext, (zeros, zeros))
            return jnp.int32(0)

        def compact_branch():
            plsc.store_scatter(cand_k, [n_cand + lane], neg1, mask=true16)
            return finish(cand_k, lambda i: cand_i[pl.ds(i * 16, 16)],
                          (n_cand + 15) // 16)

        def full_branch():
            return finish(row_v, lambda i: i * 16 + lane, nv)

        lax.cond(n_cand <= ccap, compact_branch, full_branch)

        pltpu.sync_copy(ov_v.at[pl.ds(0, cap)], vals_hbm.at[row])
        pltpu.sync_copy(oi_v.at[pl.ds(0, cap)], idx_hbm.at[row])
        return _

    lax.fori_loop(wid * rows_per_w, (wid + 1) * rows_per_w, select_row, 0)


def _make_selector(b, nb, k, cap, ccap, qpad, nc, ns, interpret=False):
    rows_per_w = b // (nc * ns)
    mesh = plsc.VectorSubcoreMesh(
        core_axis_name="c", subcore_axis_name="s",
        num_cores=nc, num_subcores=ns)
    return pl.kernel(
        functools.partial(_selector_body, nb, k, cap, ccap, qpad,
                          rows_per_w, nc),
        out_type=(jax.ShapeDtypeStruct((b, cap), jnp.int32),
                  jax.ShapeDtypeStruct((b, cap), jnp.int32)),
        mesh=mesh,
        scratch_types=[
            pltpu.VMEM((nb,), jnp.int32),
            pltpu.VMEM((qpad,), jnp.int32),
            pltpu.VMEM((ccap + 32,), jnp.int32),
            pltpu.VMEM((ccap + 32,), jnp.int32),
            pltpu.VMEM((cap + 16,), jnp.int32),
            pltpu.VMEM((cap + 16,), jnp.int32),
        ],
        compiler_params=pltpu.CompilerParams(needs_layout_passes=False),
        interpret=interpret,
    )


def kernel(pred_logits, pred_boxes, target_sizes, positive_map):
    prob, qmax = _compute_prob(pred_logits, positive_map)
    sel = _make_selector(B, NB, K, CAP, CCAP, QPAD, 2, 16)
    vbits, idxs = sel(prob.reshape(B, NB), qmax.reshape(B, QPAD))
    vals = lax.bitcast_convert_type(vbits, jnp.float32)

    scores, pos = jax.lax.top_k(vals, K)
    sidx = jnp.take_along_axis(idxs, pos, axis=1)
    topk_boxes = sidx // CPAD
    labels = sidx % CPAD

    cx, cy, w, h = (pred_boxes[..., i] for i in range(4))
    boxes = jnp.stack([cx - 0.5 * w, cy - 0.5 * h, cx + 0.5 * w, cy + 0.5 * h],
                      axis=-1)
    idx4 = jnp.repeat(topk_boxes[:, :, None], 4, axis=2)
    boxes = jnp.take_along_axis(boxes, idx4, axis=1)
    img_h = target_sizes[:, 0]
    img_w = target_sizes[:, 1]
    scale_fct = jnp.stack([img_w, img_h, img_w, img_h], axis=1)
    boxes = boxes * scale_fct[:, None, :]
    return scores, labels, boxes


# speedup vs baseline: 1.2441x; 1.2441x over previous
import functools

import jax
import jax.numpy as jnp
from jax import lax
from jax.experimental import pallas as pl
from jax.experimental.pallas import tpu as pltpu
from jax.experimental.pallas import tpu_sc as plsc

B, Q, T, C, K = 128, 900, 256, 91, 300
CPAD = 96
NB = Q * CPAD
QPAD = 1024
CAP = 512
CCAP = 8192
NEG1_BITS = -1082130432


def _prob_body(logits_ref, pm_ref, prob_ref, qmax_ref):
    x = logits_ref[0]
    sig = 1.0 / (1.0 + jnp.exp(-x))
    pm = pm_ref[...]
    prob = jax.lax.dot_general(
        sig, pm, (((1,), (1,)), ((), ())),
        preferred_element_type=jnp.float32)
    col = lax.broadcasted_iota(jnp.int32, (Q, CPAD), 1)
    bits = lax.bitcast_convert_type(prob, jnp.int32)
    bits = jnp.where(col < C, bits, NEG1_BITS)
    prob_ref[0] = bits
    qmax = jnp.max(bits, axis=1)
    qmax_ref[0, 0] = jnp.concatenate(
        [qmax, jnp.full((QPAD - Q,), NEG1_BITS, jnp.int32)])


def _compute_prob(pred_logits, pm96, nb):
    return pl.pallas_call(
        _prob_body,
        grid=(nb,),
        in_specs=[
            pl.BlockSpec((1, Q, T), lambda b: (b, 0, 0)),
            pl.BlockSpec((CPAD, T), lambda b: (0, 0)),
        ],
        out_specs=[pl.BlockSpec((1, Q, CPAD), lambda b: (b, 0, 0)),
                   pl.BlockSpec((1, 1, QPAD), lambda b: (b, 0, 0))],
        out_shape=[jax.ShapeDtypeStruct((nb, Q, CPAD), jnp.int32),
                   jax.ShapeDtypeStruct((nb, 1, QPAD), jnp.int32)],
    )(pred_logits, pm96)


def _selector_body(nb, k, cap, ccap, qpad, rows_per_w, nc, prob_hbm,
                   qmax_hbm, vals_hbm, idx_hbm, row_v, qm_v, cand_k, cand_i,
                   ov_v, oi_v):
    nv = nb // 16
    UN = 8
    lane = lax.broadcasted_iota(jnp.int32, (16,), 0)
    zeros = jnp.zeros((16,), jnp.int32)
    neg1 = jnp.full((16,), NEG1_BITS, jnp.int32)
    true16 = jnp.ones((16,), jnp.bool_)
    wid = lax.axis_index("s") * nc + lax.axis_index("c")

    def select_row(row, _):
        pltpu.sync_copy(prob_hbm.at[row], row_v)
        pltpu.sync_copy(qmax_hbm.at[row], qm_v)

        def bs_m(j, cur):
            t = cur | lax.shift_left(jnp.int32(1), 30 - j)

            def sweep(i, acc):
                return acc + (qm_v[pl.ds(i * 16, 16)] >= t).astype(jnp.int32)
            cnt = jnp.sum(lax.fori_loop(0, qpad // 16, sweep, zeros))
            return jnp.where(cnt >= k, t, cur)
        m = lax.fori_loop(0, 31, bs_m, jnp.int32(0))

        def g(t, ptrv):
            kks = []
            ms = []
            for u in range(UN):
                i = t * UN + u
                kk = row_v[pl.ds(i * 16, 16)]
                kks.append(kk)
                ms.append(kk >= m)
            anym = ms[0]
            for u in range(1, UN):
                anym = anym | ms[u]

            def do_store():
                pv = ptrv
                for u in range(UN):
                    pos = pv + plsc.cumsum(ms[u].astype(jnp.int32)) - 1
                    pos = jnp.minimum(pos, ccap + 16)
                    plsc.store_scatter(cand_k, [pos], kks[u], mask=ms[u])
                    plsc.store_scatter(cand_i, [pos], (t * UN + u) * 16 + lane,
                                       mask=ms[u])
                    pv = pv + plsc.all_reduce_population_count(ms[u])
                return pv
            return lax.cond(jnp.any(anym), do_store, lambda: ptrv)
        ptrv = lax.fori_loop(0, nv // UN, g, zeros)
        n_cand = jnp.max(ptrv)

        def initf(j, _):
            ov_v[pl.ds(j * 16, 16)] = neg1
            oi_v[pl.ds(j * 16, 16)] = zeros
            return _
        lax.fori_loop(0, (cap + 16) // 16, initf, 0)

        def finish(src_k, load_idx, ntrips):
            def bs(j, cur):
                t = cur | lax.shift_left(jnp.int32(1), 30 - j)

                def sweep(i, acc):
                    kk = src_k[pl.ds(i * 16, 16)]
                    return acc + (kk >= t).astype(jnp.int32)
                cnt = jnp.sum(lax.fori_loop(0, ntrips, sweep, zeros))
                return jnp.where(cnt >= k, t, cur)
            vk = lax.fori_loop(0, 31, bs, jnp.int32(0))

            def sweep2(i, acc):
                kk = src_k[pl.ds(i * 16, 16)]
                return acc + (kk > vk).astype(jnp.int32)
            n_gt = jnp.sum(lax.fori_loop(0, ntrips, sweep2, zeros))
            eqb = k - n_gt

            def ext(i, carry):
                ptrv, eqt = carry
                kk = src_k[pl.ds(i * 16, 16)]
                iv = load_idx(i)
                gt = kk > vk
                eq = kk == vk
                eqc = plsc.cumsum(eq.astype(jnp.int32))
                acc_eq = eq & ((eqc + eqt) <= eqb)
                accept = gt | acc_eq
                pos = ptrv + plsc.cumsum(accept.astype(jnp.int32)) - 1
                plsc.store_scatter(ov_v, [pos], kk, mask=accept)
                plsc.store_scatter(oi_v, [pos], iv, mask=accept)
                ptrv = ptrv + plsc.all_reduce_population_count(accept)
                eqt = eqt + plsc.all_reduce_population_count(acc_eq)
                return ptrv, eqt
            lax.fori_loop(0, ntrips, ext, (zeros, zeros))
            return jnp.int32(0)

        def compact_branch():
            plsc.store_scatter(cand_k, [n_cand + lane], neg1, mask=true16)
            return finish(cand_k, lambda i: cand_i[pl.ds(i * 16, 16)],
                          (n_cand + 15) // 16)

        def full_branch():
            return finish(row_v, lambda i: i * 16 + lane, nv)

        lax.cond(n_cand <= ccap, compact_branch, full_branch)

        pltpu.sync_copy(ov_v.at[pl.ds(0, cap)], vals_hbm.at[row])
        pltpu.sync_copy(oi_v.at[pl.ds(0, cap)], idx_hbm.at[row])
        return _

    lax.fori_loop(wid * rows_per_w, (wid + 1) * rows_per_w, select_row, 0)


def _make_selector(b, nb, k, cap, ccap, qpad, nc, ns, interpret=False):
    rows_per_w = b // (nc * ns)
    mesh = plsc.VectorSubcoreMesh(
        core_axis_name="c", subcore_axis_name="s",
        num_cores=nc, num_subcores=ns)
    return pl.kernel(
        functools.partial(_selector_body, nb, k, cap, ccap, qpad,
                          rows_per_w, nc),
        out_type=(jax.ShapeDtypeStruct((b, cap), jnp.int32),
                  jax.ShapeDtypeStruct((b, cap), jnp.int32)),
        mesh=mesh,
        scratch_types=[
            pltpu.VMEM((nb,), jnp.int32),
            pltpu.VMEM((qpad,), jnp.int32),
            pltpu.VMEM((ccap + 32,), jnp.int32),
            pltpu.VMEM((ccap + 32,), jnp.int32),
            pltpu.VMEM((cap + 16,), jnp.int32),
            pltpu.VMEM((cap + 16,), jnp.int32),
        ],
        compiler_params=pltpu.CompilerParams(needs_layout_passes=False),
        interpret=interpret,
    )


NCHUNK = 2


def kernel(pred_logits, pred_boxes, target_sizes, positive_map):
    pm96 = jnp.concatenate(
        [positive_map, jnp.zeros((CPAD - C, T), jnp.float32)], axis=0)
    bc = B // NCHUNK
    sel = _make_selector(bc, NB, K, CAP, CCAP, QPAD, 2, 16)
    vb_list, ix_list = [], []
    for ch in range(NCHUNK):
        logits_ch = lax.slice_in_dim(pred_logits, ch * bc, (ch + 1) * bc, 1, 0)
        prob, qmax = _compute_prob(logits_ch, pm96, bc)
        vb, ix = sel(prob.reshape(bc, NB), qmax.reshape(bc, QPAD))
        vb_list.append(vb)
        ix_list.append(ix)
    vbits = jnp.concatenate(vb_list, axis=0)
    idxs = jnp.concatenate(ix_list, axis=0)
    vals = lax.bitcast_convert_type(vbits, jnp.float32)

    scores, pos = jax.lax.top_k(vals, K)
    sidx = jnp.take_along_axis(idxs, pos, axis=1)
    topk_boxes = sidx // CPAD
    labels = sidx % CPAD

    idx4 = jnp.repeat(topk_boxes[:, :, None], 4, axis=2)
    bsel = jnp.take_along_axis(pred_boxes, idx4, axis=1)
    cx, cy, w, h = (bsel[..., i] for i in range(4))
    boxes = jnp.stack([cx - 0.5 * w, cy - 0.5 * h, cx + 0.5 * w, cy + 0.5 * h],
                      axis=-1)
    img_h = target_sizes[:, 0]
    img_w = target_sizes[:, 1]
    scale_fct = jnp.stack([img_w, img_h, img_w, img_h], axis=1)
    boxes = boxes * scale_fct[:, None, :]
    return scores, labels, boxes


# speedup vs baseline: 1.2553x; 1.0090x over previous
import functools

import jax
import jax.numpy as jnp
from jax import lax
from jax.experimental import pallas as pl
from jax.experimental.pallas import tpu as pltpu
from jax.experimental.pallas import tpu_sc as plsc

B, Q, T, C, K = 128, 900, 256, 91, 300
CPAD = 96
NB = Q * CPAD
QPAD = 1024
CAP = 512
CCAP = 8192
NEG1_BITS = -1082130432


def _prob_body(logits_ref, pm_ref, prob_ref, qmax_ref):
    x = logits_ref[0]
    sig = 1.0 / (1.0 + jnp.exp(-x))
    pm = pm_ref[...]
    prob = jax.lax.dot_general(
        sig, pm, (((1,), (1,)), ((), ())),
        preferred_element_type=jnp.float32)
    col = lax.broadcasted_iota(jnp.int32, (Q, CPAD), 1)
    bits = lax.bitcast_convert_type(prob, jnp.int32)
    bits = jnp.where(col < C, bits, NEG1_BITS)
    prob_ref[0] = bits
    qmax = jnp.max(bits, axis=1)
    qmax_ref[0, 0] = jnp.concatenate(
        [qmax, jnp.full((QPAD - Q,), NEG1_BITS, jnp.int32)])


def _compute_prob(pred_logits, pm96, nb):
    return pl.pallas_call(
        _prob_body,
        grid=(nb,),
        in_specs=[
            pl.BlockSpec((1, Q, T), lambda b: (b, 0, 0)),
            pl.BlockSpec((CPAD, T), lambda b: (0, 0)),
        ],
        out_specs=[pl.BlockSpec((1, Q, CPAD), lambda b: (b, 0, 0)),
                   pl.BlockSpec((1, 1, QPAD), lambda b: (b, 0, 0))],
        out_shape=[jax.ShapeDtypeStruct((nb, Q, CPAD), jnp.int32),
                   jax.ShapeDtypeStruct((nb, 1, QPAD), jnp.int32)],
    )(pred_logits, pm96)


def _selector_body(nb, k, cap, ccap, qpad, rows_per_w, nc, prob_hbm,
                   qmax_hbm, vals_hbm, idx_hbm, row_v, qm_v, cand_k, cand_i,
                   ov_v, oi_v):
    nv = nb // 16
    UN = 8
    lane = lax.broadcasted_iota(jnp.int32, (16,), 0)
    zeros = jnp.zeros((16,), jnp.int32)
    neg1 = jnp.full((16,), NEG1_BITS, jnp.int32)
    true16 = jnp.ones((16,), jnp.bool_)
    wid = lax.axis_index("s") * nc + lax.axis_index("c")

    def select_row(row, _):
        pltpu.sync_copy(prob_hbm.at[row], row_v)
        pltpu.sync_copy(qmax_hbm.at[row], qm_v)

        def bs_m(j, cur):
            t = cur | lax.shift_left(jnp.int32(1), 30 - j)

            def sweep(i, acc):
                return acc + (qm_v[pl.ds(i * 16, 16)] >= t).astype(jnp.int32)
            cnt = jnp.sum(lax.fori_loop(0, qpad // 16, sweep, zeros))
            return jnp.where(cnt >= k, t, cur)
        m = lax.fori_loop(0, 31, bs_m, jnp.int32(0))

        def g(t, ptrv):
            kks = []
            ms = []
            for u in range(UN):
                i = t * UN + u
                kk = row_v[pl.ds(i * 16, 16)]
                kks.append(kk)
                ms.append(kk >= m)
            anym = ms[0]
            for u in range(1, UN):
                anym = anym | ms[u]

            def do_store():
                pv = ptrv
                for u in range(UN):
                    pos = pv + plsc.cumsum(ms[u].astype(jnp.int32)) - 1
                    pos = jnp.minimum(pos, ccap + 16)
                    plsc.store_scatter(cand_k, [pos], kks[u], mask=ms[u])
                    plsc.store_scatter(cand_i, [pos], (t * UN + u) * 16 + lane,
                                       mask=ms[u])
                    pv = pv + plsc.all_reduce_population_count(ms[u])
                return pv
            return lax.cond(jnp.any(anym), do_store, lambda: ptrv)
        ptrv = lax.fori_loop(0, nv // UN, g, zeros)
        n_cand = jnp.max(ptrv)

        def initf(j, _):
            ov_v[pl.ds(j * 16, 16)] = neg1
            oi_v[pl.ds(j * 16, 16)] = zeros
            return _
        lax.fori_loop(0, (cap + 16) // 16, initf, 0)

        def finish(src_k, load_idx, ntrips):
            def bs(j, cur):
                t = cur | lax.shift_left(jnp.int32(1), 30 - j)

                def sweep(i, acc):
                    kk = src_k[pl.ds(i * 16, 16)]
                    return acc + (kk >= t).astype(jnp.int32)
                cnt = jnp.sum(lax.fori_loop(0, ntrips, sweep, zeros))
                return jnp.where(cnt >= k, t, cur)
            vk = lax.fori_loop(0, 31, bs, jnp.int32(0))

            def sweep2(i, acc):
                kk = src_k[pl.ds(i * 16, 16)]
                return acc + (kk > vk).astype(jnp.int32)
            n_gt = jnp.sum(lax.fori_loop(0, ntrips, sweep2, zeros))
            eqb = k - n_gt

            def ext(i, carry):
                ptrv, eqt = carry
                kk = src_k[pl.ds(i * 16, 16)]
                iv = load_idx(i)
                gt = kk > vk
                eq = kk == vk
                eqc = plsc.cumsum(eq.astype(jnp.int32))
                acc_eq = eq & ((eqc + eqt) <= eqb)
                accept = gt | acc_eq
                pos = ptrv + plsc.cumsum(accept.astype(jnp.int32)) - 1
                plsc.store_scatter(ov_v, [pos], kk, mask=accept)
                plsc.store_scatter(oi_v, [pos], iv, mask=accept)
                ptrv = ptrv + plsc.all_reduce_population_count(accept)
                eqt = eqt + plsc.all_reduce_population_count(acc_eq)
                return ptrv, eqt
            lax.fori_loop(0, ntrips, ext, (zeros, zeros))
            return jnp.int32(0)

        def compact_branch():
            plsc.store_scatter(cand_k, [n_cand + lane], neg1, mask=true16)
            return finish(cand_k, lambda i: cand_i[pl.ds(i * 16, 16)],
                          (n_cand + 15) // 16)

        def full_branch():
            return finish(row_v, lambda i: i * 16 + lane, nv)

        lax.cond(n_cand <= ccap, compact_branch, full_branch)

        pltpu.sync_copy(ov_v.at[pl.ds(0, cap)], vals_hbm.at[row])
        pltpu.sync_copy(oi_v.at[pl.ds(0, cap)], idx_hbm.at[row])
        return _

    lax.fori_loop(wid * rows_per_w, (wid + 1) * rows_per_w, select_row, 0)


def _make_selector(b, nb, k, cap, ccap, qpad, nc, ns, interpret=False):
    rows_per_w = b // (nc * ns)
    mesh = plsc.VectorSubcoreMesh(
        core_axis_name="c", subcore_axis_name="s",
        num_cores=nc, num_subcores=ns)
    return pl.kernel(
        functools.partial(_selector_body, nb, k, cap, ccap, qpad,
                          rows_per_w, nc),
        out_type=(jax.ShapeDtypeStruct((b, cap), jnp.int32),
                  jax.ShapeDtypeStruct((b, cap), jnp.int32)),
        mesh=mesh,
        scratch_types=[
            pltpu.VMEM((nb,), jnp.int32),
            pltpu.VMEM((qpad,), jnp.int32),
            pltpu.VMEM((ccap + 32,), jnp.int32),
            pltpu.VMEM((ccap + 32,), jnp.int32),
            pltpu.VMEM((cap + 16,), jnp.int32),
            pltpu.VMEM((cap + 16,), jnp.int32),
        ],
        compiler_params=pltpu.CompilerParams(needs_layout_passes=False),
        interpret=interpret,
    )


NCHUNK = 4


def kernel(pred_logits, pred_boxes, target_sizes, positive_map):
    pm96 = jnp.concatenate(
        [positive_map, jnp.zeros((CPAD - C, T), jnp.float32)], axis=0)
    bc = B // NCHUNK
    sel = _make_selector(bc, NB, K, CAP, CCAP, QPAD, 2, 16)
    vb_list, ix_list = [], []
    for ch in range(NCHUNK):
        logits_ch = lax.slice_in_dim(pred_logits, ch * bc, (ch + 1) * bc, 1, 0)
        prob, qmax = _compute_prob(logits_ch, pm96, bc)
        vb, ix = sel(prob.reshape(bc, NB), qmax.reshape(bc, QPAD))
        vb_list.append(vb)
        ix_list.append(ix)
    vbits = jnp.concatenate(vb_list, axis=0)
    idxs = jnp.concatenate(ix_list, axis=0)
    vals = lax.bitcast_convert_type(vbits, jnp.float32)

    scores, pos = jax.lax.top_k(vals, K)
    sidx = jnp.take_along_axis(idxs, pos, axis=1)
    topk_boxes = sidx // CPAD
    labels = sidx % CPAD

    idx4 = jnp.repeat(topk_boxes[:, :, None], 4, axis=2)
    bsel = jnp.take_along_axis(pred_boxes, idx4, axis=1)
    cx, cy, w, h = (bsel[..., i] for i in range(4))
    boxes = jnp.stack([cx - 0.5 * w, cy - 0.5 * h, cx + 0.5 * w, cy + 0.5 * h],
                      axis=-1)
    img_h = target_sizes[:, 0]
    img_w = target_sizes[:, 1]
    scale_fct = jnp.stack([img_w, img_h, img_w, img_h], axis=1)
    boxes = boxes * scale_fct[:, None, :]
    return scores, labels, boxes


# speedup vs baseline: 1.4818x; 1.1804x over previous
import functools

import jax
import jax.numpy as jnp
from jax import lax
from jax.experimental import pallas as pl
from jax.experimental.pallas import tpu as pltpu
from jax.experimental.pallas import tpu_sc as plsc

B, Q, T, C, K = 128, 900, 256, 91, 300
CPAD = 96
NB = Q * CPAD
QPAD = 1024
CAP = 512
CCAP = 8192
NEG1_BITS = -1082130432


def _prob_body(logits_ref, pm_ref, prob_ref, qmax_ref):
    x = logits_ref[0]
    sig = 1.0 / (1.0 + jnp.exp(-x))
    pm = pm_ref[...]
    prob = jax.lax.dot_general(
        sig, pm, (((1,), (1,)), ((), ())),
        preferred_element_type=jnp.float32)
    col = lax.broadcasted_iota(jnp.int32, (Q, CPAD), 1)
    bits = lax.bitcast_convert_type(prob, jnp.int32)
    bits = jnp.where(col < C, bits, NEG1_BITS)
    prob_ref[0] = bits
    qmax = jnp.max(bits, axis=1)
    qmax_ref[0, 0] = jnp.concatenate(
        [qmax, jnp.full((QPAD - Q,), NEG1_BITS, jnp.int32)])


def _compute_prob(pred_logits, pm96, nb, b0):
    return pl.pallas_call(
        _prob_body,
        grid=(nb,),
        in_specs=[
            pl.BlockSpec((1, Q, T), lambda b: (b + b0, 0, 0)),
            pl.BlockSpec((CPAD, T), lambda b: (0, 0)),
        ],
        out_specs=[pl.BlockSpec((1, Q, CPAD), lambda b: (b, 0, 0)),
                   pl.BlockSpec((1, 1, QPAD), lambda b: (b, 0, 0))],
        out_shape=[jax.ShapeDtypeStruct((nb, Q, CPAD), jnp.int32),
                   jax.ShapeDtypeStruct((nb, 1, QPAD), jnp.int32)],
    )(pred_logits, pm96)


def _selector_body(nb, k, cap, ccap, qpad, rows_per_w, nc, prob_hbm,
                   qmax_hbm, vals_hbm, idx_hbm, row_v, qm_v, cand_k, cand_i,
                   ov_v, oi_v):
    nv = nb // 16
    UN = 8
    lane = lax.broadcasted_iota(jnp.int32, (16,), 0)
    zeros = jnp.zeros((16,), jnp.int32)
    neg1 = jnp.full((16,), NEG1_BITS, jnp.int32)
    true16 = jnp.ones((16,), jnp.bool_)
    wid = lax.axis_index("s") * nc + lax.axis_index("c")

    def select_row(row, _):
        pltpu.sync_copy(prob_hbm.at[row], row_v)
        pltpu.sync_copy(qmax_hbm.at[row], qm_v)

        def bs_m(j, cur):
            t = cur | lax.shift_left(jnp.int32(1), 30 - j)

            def sweep(i, acc):
                return acc + (qm_v[pl.ds(i * 16, 16)] >= t).astype(jnp.int32)
            cnt = jnp.sum(lax.fori_loop(0, qpad // 16, sweep, zeros))
            return jnp.where(cnt >= k, t, cur)
        m = lax.fori_loop(0, 31, bs_m, jnp.int32(0))

        def g(t, ptrv):
            kks = []
            ms = []
            for u in range(UN):
                i = t * UN + u
                kk = row_v[pl.ds(i * 16, 16)]
                kks.append(kk)
                ms.append(kk >= m)
            anym = ms[0]
            for u in range(1, UN):
                anym = anym | ms[u]

            def do_store():
                pv = ptrv
                for u in range(UN):
                    pos = pv + plsc.cumsum(ms[u].astype(jnp.int32)) - 1
                    pos = jnp.minimum(pos, ccap + 16)
                    plsc.store_scatter(cand_k, [pos], kks[u], mask=ms[u])
                    plsc.store_scatter(cand_i, [pos], (t * UN + u) * 16 + lane,
                                       mask=ms[u])
                    pv = pv + plsc.all_reduce_population_count(ms[u])
                return pv
            return lax.cond(jnp.any(anym), do_store, lambda: ptrv)
        ptrv = lax.fori_loop(0, nv // UN, g, zeros)
        n_cand = jnp.max(ptrv)

        def initf(j, _):
            ov_v[pl.ds(j * 16, 16)] = neg1
            oi_v[pl.ds(j * 16, 16)] = zeros
            return _
        lax.fori_loop(0, (cap + 16) // 16, initf, 0)

        def finish(src_k, load_idx, ntrips):
            def bs(j, cur):
                t = cur | lax.shift_left(jnp.int32(1), 30 - j)

                def sweep(i, acc):
                    kk = src_k[pl.ds(i * 16, 16)]
                    return acc + (kk >= t).astype(jnp.int32)
                cnt = jnp.sum(lax.fori_loop(0, ntrips, sweep, zeros))
                return jnp.where(cnt >= k, t, cur)
            vk = lax.fori_loop(0, 31, bs, jnp.int32(0))

            def sweep2(i, acc):
                kk = src_k[pl.ds(i * 16, 16)]
                return acc + (kk > vk).astype(jnp.int32)
            n_gt = jnp.sum(lax.fori_loop(0, ntrips, sweep2, zeros))
            eqb = k - n_gt

            def ext(i, carry):
                ptrv, eqt = carry
                kk = src_k[pl.ds(i * 16, 16)]
                iv = load_idx(i)
                gt = kk > vk
                eq = kk == vk
                eqc = plsc.cumsum(eq.astype(jnp.int32))
                acc_eq = eq & ((eqc + eqt) <= eqb)
                accept = gt | acc_eq
                pos = ptrv + plsc.cumsum(accept.astype(jnp.int32)) - 1
                plsc.store_scatter(ov_v, [pos], kk, mask=accept)
                plsc.store_scatter(oi_v, [pos], iv, mask=accept)
                ptrv = ptrv + plsc.all_reduce_population_count(accept)
                eqt = eqt + plsc.all_reduce_population_count(acc_eq)
                return ptrv, eqt
            lax.fori_loop(0, ntrips, ext, (zeros, zeros))
            return jnp.int32(0)

        def compact_branch():
            plsc.store_scatter(cand_k, [n_cand + lane], neg1, mask=true16)
            return finish(cand_k, lambda i: cand_i[pl.ds(i * 16, 16)],
                          (n_cand + 15) // 16)

        def full_branch():
            return finish(row_v, lambda i: i * 16 + lane, nv)

        lax.cond(n_cand <= ccap, compact_branch, full_branch)

        pltpu.sync_copy(ov_v.at[pl.ds(0, cap)], vals_hbm.at[row])
        pltpu.sync_copy(oi_v.at[pl.ds(0, cap)], idx_hbm.at[row])
        return _

    lax.fori_loop(wid * rows_per_w, (wid + 1) * rows_per_w, select_row, 0)


def _make_selector(b, nb, k, cap, ccap, qpad, nc, ns, interpret=False):
    rows_per_w = b // (nc * ns)
    mesh = plsc.VectorSubcoreMesh(
        core_axis_name="c", subcore_axis_name="s",
        num_cores=nc, num_subcores=ns)
    return pl.kernel(
        functools.partial(_selector_body, nb, k, cap, ccap, qpad,
                          rows_per_w, nc),
        out_type=(jax.ShapeDtypeStruct((b, cap), jnp.int32),
                  jax.ShapeDtypeStruct((b, cap), jnp.int32)),
        mesh=mesh,
        scratch_types=[
            pltpu.VMEM((nb,), jnp.int32),
            pltpu.VMEM((qpad,), jnp.int32),
            pltpu.VMEM((ccap + 32,), jnp.int32),
            pltpu.VMEM((ccap + 32,), jnp.int32),
            pltpu.VMEM((cap + 16,), jnp.int32),
            pltpu.VMEM((cap + 16,), jnp.int32),
        ],
        compiler_params=pltpu.CompilerParams(needs_layout_passes=False),
        interpret=interpret,
    )


NCHUNK = 4


def kernel(pred_logits, pred_boxes, target_sizes, positive_map):
    pm96 = jnp.concatenate(
        [positive_map, jnp.zeros((CPAD - C, T), jnp.float32)], axis=0)
    bc = B // NCHUNK
    sel = _make_selector(bc, NB, K, CAP, CCAP, QPAD, 2, 16)
    vb_list, ix_list = [], []
    for ch in range(NCHUNK):
        prob, qmax = _compute_prob(pred_logits, pm96, bc, ch * bc)
        vb, ix = sel(prob.reshape(bc, NB), qmax.reshape(bc, QPAD))
        vb_list.append(vb)
        ix_list.append(ix)
    vbits = jnp.concatenate(vb_list, axis=0)
    idxs = jnp.concatenate(ix_list, axis=0)
    vals = lax.bitcast_convert_type(vbits, jnp.float32)

    scores, pos = jax.lax.top_k(vals, K)
    sidx = jnp.take_along_axis(idxs, pos, axis=1)
    topk_boxes = sidx // CPAD
    labels = sidx % CPAD

    idx4 = jnp.repeat(topk_boxes[:, :, None], 4, axis=2)
    bsel = jnp.take_along_axis(pred_boxes, idx4, axis=1)
    cx, cy, w, h = (bsel[..., i] for i in range(4))
    boxes = jnp.stack([cx - 0.5 * w, cy - 0.5 * h, cx + 0.5 * w, cy + 0.5 * h],
                      axis=-1)
    img_h = target_sizes[:, 0]
    img_w = target_sizes[:, 1]
    scale_fct = jnp.stack([img_w, img_h, img_w, img_h], axis=1)
    boxes = boxes * scale_fct[:, None, :]
    return scores, labels, boxes
